# Initial kernel scaffold; baseline (speedup 1.0000x reference)
import jax
import jax.numpy as jnp
from jax.experimental import pallas as pl


def kernel(q, k, v, W_q, b_q, W_k, b_k, W_v, b_v, W_c, b_c):
    raise NotImplementedError("write your pallas kernel here")


# speedup vs baseline: 2.6750x; 2.6750x over previous
import functools

import jax
import jax.numpy as jnp
from jax import lax
from jax.experimental import pallas as pl
from jax.experimental.pallas import tpu as pltpu
from jax.experimental.pallas import tpu_sc as plsc

L = 2048
H = 12
DM = 768
DT = 64
NUM = 100
BL = 64


def _proj_body(x_ref, w_ref, b_ref, o_ref):
    o_ref[...] = lax.dot_general(
        x_ref[...], w_ref[...], (((1,), (1,)), ((), ())),
        preferred_element_type=jnp.float32) + b_ref[...]


def _project(x, w, b):
    return pl.pallas_call(
        _proj_body,
        out_shape=jax.ShapeDtypeStruct((L, DM), jnp.float32),
    )(x, w, b)


_CR = 8
_CL = 8


def _masks(shape, j, k, asc_parity):
    lane = lax.broadcasted_iota(jnp.int32, shape, 2)
    is_lower = (lane & j) == 0
    if k is None:
        if asc_parity is None:
            asc = jnp.full(shape, True)
        else:
            blk = lax.broadcasted_iota(jnp.int32, shape, 1)
            asc = (blk & 1) == 1
    else:
        blk = lax.broadcasted_iota(jnp.int32, shape, 1)
        asc = jnp.logical_xor((lane & k) != 0, (blk & 1) == 1)
    return is_lower, asc


def _cx(v, ix, j, is_lower, asc):
    not_lower = jnp.logical_not(is_lower)
    pv = jnp.where(is_lower, jnp.roll(v, -j, axis=-1), jnp.roll(v, j, axis=-1))
    pix = jnp.where(is_lower, jnp.roll(ix, -j, axis=-1), jnp.roll(ix, j, axis=-1))
    gt = v > pv
    lt = v < pv
    use_gt = is_lower == asc
    keep = jnp.logical_not(jnp.logical_or(
        jnp.logical_and(use_gt, gt),
        jnp.logical_and(jnp.logical_not(use_gt), lt)))
    return jnp.where(keep, v, pv), jnp.where(keep, ix, pix)


def _topk100_asc_idx(p):
    r = p.shape[0]
    nb = 16
    v = p.reshape(r, nb, 128)
    lane = lax.broadcasted_iota(jnp.int32, (r, nb, 128), 2)
    blk = lax.broadcasted_iota(jnp.int32, (r, nb, 128), 1)
    ix = lane + 128 * blk
    k = 2
    while k <= 128:
        j = k // 2
        while j >= 1:
            is_lower, asc = _masks((r, nb, 128), j, k, 0)
            v, ix = _cx(v, ix, j, is_lower, asc)
            j //= 2
        k *= 2
    while nb > 1:
        v4 = v.reshape(r, nb // 2, 2, 128)
        ix4 = ix.reshape(r, nb // 2, 2, 128)
        t = v4[:, :, 0, :] >= v4[:, :, 1, :]
        v = jnp.where(t, v4[:, :, 0, :], v4[:, :, 1, :])
        ix = jnp.where(t, ix4[:, :, 0, :], ix4[:, :, 1, :])
        nb //= 2
        j = 64
        while j >= 1:
            is_lower, asc = _masks((r, nb, 128), j, None,
                                   0 if nb > 1 else None)
            v, ix = _cx(v, ix, j, is_lower, asc)
            j //= 2
    return ix[:, 0, 128 - NUM:]


def _score_topk_body(q_ref, k_ref, o_ref, p_ref):
    for h in range(H):
        p_ref[h] = lax.dot_general(
            q_ref[h], k_ref[h], (((1,), (0,)), ((), ())),
            preferred_element_type=jnp.float32) * 0.125

    def softmax_step(c, _):
        s = p_ref[:, pl.ds(c * _CL, _CL), :]
        m = jnp.max(s, axis=0, keepdims=True)
        e = jnp.exp(s - m)
        p_ref[:, pl.ds(c * _CL, _CL), :] = e / jnp.sum(e, axis=0, keepdims=True)
        return ()

    lax.fori_loop(0, BL // _CL, softmax_step, (), unroll=False)

    chunks_per_h = BL // _CR

    def topk_step(c, _):
        h = c // chunks_per_h
        l0 = (c % chunks_per_h) * _CR
        pc = p_ref[h, pl.ds(l0, _CR), :]
        idx = _topk100_asc_idx(pc)
        o_ref[h, pl.ds(l0, _CR), :] = idx + 2048 * h
        return ()

    lax.fori_loop(0, H * chunks_per_h, topk_step, (), unroll=False)


def _score_topk(qh, kt):
    return pl.pallas_call(
        _score_topk_body,
        grid=(L // BL,),
        in_specs=[
            pl.BlockSpec((H, BL, DT), lambda i: (0, i, 0)),
            pl.BlockSpec((H, DT, L), lambda i: (0, 0, 0)),
        ],
        out_specs=pl.BlockSpec((H, BL, NUM), lambda i: (0, i, 0)),
        out_shape=jax.ShapeDtypeStruct((H, L, NUM), jnp.int32),
        scratch_shapes=[pltpu.VMEM((H, BL, L), jnp.float32)],
    )(qh, kt)


_NROW = H * L * NUM
_NCHUNK = _NROW // 128
_FIRE = 4


def _pad_body(x_ref, o_ref):
    o_ref[:, :DT] = x_ref[...]
    o_ref[:, DT:] = jnp.zeros_like(o_ref[:, DT:])


def _pad_table(v2):
    return pl.pallas_call(
        _pad_body,
        out_shape=jax.ShapeDtypeStruct((H * L, 128), jnp.float32),
    )(v2)


_NC = 2
_NS = 16


def _make_gather():
    nw = _NC * _NS
    chunks_per_w = _NCHUNK // nw
    iters = chunks_per_w // _FIRE
    mesh = plsc.VectorSubcoreMesh(core_axis_name="c", subcore_axis_name="s")

    @functools.partial(
        pl.kernel, mesh=mesh,
        out_type=jax.ShapeDtypeStruct((_NCHUNK, 128, 128), jnp.float32),
        scratch_types=[
            pltpu.VMEM((_FIRE, 128), jnp.int32),
            pltpu.VMEM((_FIRE, 128, 128), jnp.float32),
            pltpu.SemaphoreType.DMA,
        ],
    )
    def gather(table_hbm, idx_hbm, out_hbm, idx_v, rows_v, sem):
        wid = lax.axis_index("s") * _NC + lax.axis_index("c")
        base = wid * chunks_per_w

        def step(t, _):
            row0 = base + t * _FIRE
            pltpu.sync_copy(idx_hbm.at[pl.ds(row0, _FIRE)], idx_v)
            cps = [
                pltpu.async_copy(table_hbm.at[idx_v.at[c]], rows_v.at[c], sem)
                for c in range(_FIRE)
            ]
            for cp in cps:
                cp.wait()
            pltpu.sync_copy(rows_v, out_hbm.at[pl.ds(row0, _FIRE)])
            return ()

        lax.fori_loop(0, iters, step, (), unroll=False)

    return gather


_gather_fn = None


def _gather(table, idx2d):
    global _gather_fn
    if _gather_fn is None:
        _gather_fn = _make_gather()
    return _gather_fn(table, idx2d)


_BM = 1024


def _out_body(g_ref, w_ref, b_ref, o_ref):
    gv = g_ref[...]
    acc = jnp.broadcast_to(b_ref[...], (_BM, DM))
    for j in range(H):
        acc = acc + lax.dot_general(
            gv[:, j, :DT], w_ref[j], (((1,), (0,)), ((), ())),
            preferred_element_type=jnp.float32)
    o_ref[...] = acc


def _out_matmul(g3, wstack, b):
    m = g3.shape[0]
    return pl.pallas_call(
        _out_body,
        grid=(m // _BM,),
        in_specs=[
            pl.BlockSpec((_BM, H, 128), lambda i: (i, 0, 0)),
            pl.BlockSpec((H, DT, DM), lambda i: (0, 0, 0)),
            pl.BlockSpec((1, DM), lambda i: (0, 0)),
        ],
        out_specs=pl.BlockSpec((_BM, DM), lambda i: (i, 0)),
        out_shape=jax.ShapeDtypeStruct((m, DM), jnp.float32),
    )(g3, wstack, b)


def kernel(q, k, v, W_q, b_q, W_k, b_k, W_v, b_v, W_c, b_c):
    q2, k2, v2 = q[:, 0, :], k[:, 0, :], v[:, 0, :]
    qp = _project(q2, W_q, b_q.reshape(1, DM))
    kp = _project(k2, W_k, b_k.reshape(1, DM))
    vp = _project(v2, W_v, b_v.reshape(1, DM))
    g_idx = _score_topk(qp.reshape(H, L, DT), kp.reshape(H, DT, L))
    table = _pad_table(vp.reshape(H * L, DT))
    g3 = _gather(table, g_idx.reshape(_NCHUNK, 128))
    wstack = W_c.T.reshape(H, DT, DM)
    out2d = _out_matmul(g3.reshape(L * NUM, H, 128), wstack,
                        b_c.reshape(1, DM))
    return out2d.reshape(1, L, NUM, DM)


# speedup vs baseline: 2.8694x; 1.0727x over previous
import functools

import jax
import jax.numpy as jnp
from jax import lax
from jax.experimental import pallas as pl
from jax.experimental.pallas import tpu as pltpu
from jax.experimental.pallas import tpu_sc as plsc

L = 2048
H = 12
DM = 768
DT = 64
NUM = 100
BL = 64


def _proj_body(x_ref, w_ref, b_ref, o_ref):
    o_ref[...] = lax.dot_general(
        x_ref[...], w_ref[...], (((1,), (1,)), ((), ())),
        preferred_element_type=jnp.float32) + b_ref[...]


def _project(x, w, b):
    return pl.pallas_call(
        _proj_body,
        out_shape=jax.ShapeDtypeStruct((L, DM), jnp.float32),
    )(x, w, b)


_CR = 8
_CL = 8


def _masks(shape, j, k, asc_parity):
    lane = lax.broadcasted_iota(jnp.int32, shape, 2)
    low = lane & j
    is_lower = low == 0
    if k is None:
        if asc_parity is None:
            take_max = jnp.logical_not(is_lower)
        else:
            blk = lax.broadcasted_iota(jnp.int32, shape, 1)
            take_max = jnp.logical_xor(is_lower, (blk & 1) == 1)
    else:
        blk = lax.broadcasted_iota(jnp.int32, shape, 1)
        asc = jnp.logical_xor((lane & k) != 0, (blk & 1) == 1)
        take_max = jnp.logical_xor(is_lower, asc)
    return is_lower, take_max


def _cx(v, ix, j, is_lower, take_max):
    w = jnp.where(is_lower, jnp.roll(v, -j, axis=-1), jnp.roll(v, j, axis=-1))
    nv = jnp.where(take_max, jnp.maximum(v, w), jnp.minimum(v, w))
    pix = jnp.where(is_lower, jnp.roll(ix, -j, axis=-1), jnp.roll(ix, j, axis=-1))
    nix = jnp.where(nv == v, ix, pix)
    return nv, nix


def _topk100_asc_idx(p):
    r = p.shape[0]
    nb = 16
    v = p.reshape(r, nb, 128)
    lane = lax.broadcasted_iota(jnp.int32, (r, nb, 128), 2)
    blk = lax.broadcasted_iota(jnp.int32, (r, nb, 128), 1)
    ix = lane + 128 * blk
    k = 2
    while k <= 128:
        j = k // 2
        while j >= 1:
            is_lower, asc = _masks((r, nb, 128), j, k, 0)
            v, ix = _cx(v, ix, j, is_lower, asc)
            j //= 2
        k *= 2
    while nb > 1:
        v4 = v.reshape(r, nb // 2, 2, 128)
        ix4 = ix.reshape(r, nb // 2, 2, 128)
        t = v4[:, :, 0, :] >= v4[:, :, 1, :]
        v = jnp.where(t, v4[:, :, 0, :], v4[:, :, 1, :])
        ix = jnp.where(t, ix4[:, :, 0, :], ix4[:, :, 1, :])
        nb //= 2
        j = 64
        while j >= 1:
            is_lower, asc = _masks((r, nb, 128), j, None,
                                   0 if nb > 1 else None)
            v, ix = _cx(v, ix, j, is_lower, asc)
            j //= 2
    return ix[:, 0, 128 - NUM:]


def _score_topk_body(q_ref, k_ref, o_ref, p_ref):
    for h in range(H):
        p_ref[h] = lax.dot_general(
            q_ref[h], k_ref[h], (((1,), (0,)), ((), ())),
            preferred_element_type=jnp.float32) * 0.125

    def softmax_step(c, _):
        s = p_ref[:, pl.ds(c * _CL, _CL), :]
        m = jnp.max(s, axis=0, keepdims=True)
        e = jnp.exp(s - m)
        p_ref[:, pl.ds(c * _CL, _CL), :] = e / jnp.sum(e, axis=0, keepdims=True)
        return ()

    lax.fori_loop(0, BL // _CL, softmax_step, (), unroll=False)

    chunks_per_h = BL // _CR

    def topk_step(c, _):
        h = c // chunks_per_h
        l0 = (c % chunks_per_h) * _CR
        pc = p_ref[h, pl.ds(l0, _CR), :]
        idx = _topk100_asc_idx(pc)
        o_ref[h, pl.ds(l0, _CR), :] = idx + 2048 * h
        return ()

    lax.fori_loop(0, H * chunks_per_h, topk_step, (), unroll=False)


def _score_topk(qh, kt):
    return pl.pallas_call(
        _score_topk_body,
        grid=(L // BL,),
        in_specs=[
            pl.BlockSpec((H, BL, DT), lambda i: (0, i, 0)),
            pl.BlockSpec((H, DT, L), lambda i: (0, 0, 0)),
        ],
        out_specs=pl.BlockSpec((H, BL, NUM), lambda i: (0, i, 0)),
        out_shape=jax.ShapeDtypeStruct((H, L, NUM), jnp.int32),
        scratch_shapes=[pltpu.VMEM((H, BL, L), jnp.float32)],
    )(qh, kt)


_NROW = H * L * NUM
_NCHUNK = _NROW // 128
_FIRE = 4


def _pad_body(x_ref, o_ref):
    o_ref[:, :DT] = x_ref[...]
    o_ref[:, DT:] = jnp.zeros_like(o_ref[:, DT:])


def _pad_table(v2):
    return pl.pallas_call(
        _pad_body,
        out_shape=jax.ShapeDtypeStruct((H * L, 128), jnp.float32),
    )(v2)


_NC = 2
_NS = 16


def _make_gather():
    nw = _NC * _NS
    chunks_per_w = _NCHUNK // nw
    iters = chunks_per_w // _FIRE
    mesh = plsc.VectorSubcoreMesh(core_axis_name="c", subcore_axis_name="s")

    @functools.partial(
        pl.kernel, mesh=mesh,
        out_type=jax.ShapeDtypeStruct((_NCHUNK, 128, 128), jnp.float32),
        scratch_types=[
            pltpu.VMEM((_FIRE, 128), jnp.int32),
            pltpu.VMEM((_FIRE, 128, 128), jnp.float32),
            pltpu.SemaphoreType.DMA,
        ],
    )
    def gather(table_hbm, idx_hbm, out_hbm, idx_v, rows_v, sem):
        wid = lax.axis_index("s") * _NC + lax.axis_index("c")
        base = wid * chunks_per_w

        def step(t, _):
            row0 = base + t * _FIRE
            pltpu.sync_copy(idx_hbm.at[pl.ds(row0, _FIRE)], idx_v)
            cps = [
                pltpu.async_copy(table_hbm.at[idx_v.at[c]], rows_v.at[c], sem)
                for c in range(_FIRE)
            ]
            for cp in cps:
                cp.wait()
            pltpu.sync_copy(rows_v, out_hbm.at[pl.ds(row0, _FIRE)])
            return ()

        lax.fori_loop(0, iters, step, (), unroll=False)

    return gather


_gather_fn = None


def _gather(table, idx2d):
    global _gather_fn
    if _gather_fn is None:
        _gather_fn = _make_gather()
    return _gather_fn(table, idx2d)


_BM = 1024


def _out_body(g_ref, w_ref, b_ref, o_ref):
    gv = g_ref[...]
    acc = jnp.broadcast_to(b_ref[...], (_BM, DM))
    for j in range(H):
        acc = acc + lax.dot_general(
            gv[:, j, :DT], w_ref[j], (((1,), (0,)), ((), ())),
            preferred_element_type=jnp.float32)
    o_ref[...] = acc


def _out_matmul(g3, wstack, b):
    m = g3.shape[0]
    return pl.pallas_call(
        _out_body,
        grid=(m // _BM,),
        in_specs=[
            pl.BlockSpec((_BM, H, 128), lambda i: (i, 0, 0)),
            pl.BlockSpec((H, DT, DM), lambda i: (0, 0, 0)),
            pl.BlockSpec((1, DM), lambda i: (0, 0)),
        ],
        out_specs=pl.BlockSpec((_BM, DM), lambda i: (i, 0)),
        out_shape=jax.ShapeDtypeStruct((m, DM), jnp.float32),
    )(g3, wstack, b)


def kernel(q, k, v, W_q, b_q, W_k, b_k, W_v, b_v, W_c, b_c):
    q2, k2, v2 = q[:, 0, :], k[:, 0, :], v[:, 0, :]
    qp = _project(q2, W_q, b_q.reshape(1, DM))
    kp = _project(k2, W_k, b_k.reshape(1, DM))
    vp = _project(v2, W_v, b_v.reshape(1, DM))
    g_idx = _score_topk(qp.reshape(H, L, DT), kp.reshape(H, DT, L))
    table = _pad_table(vp.reshape(H * L, DT))
    g3 = _gather(table, g_idx.reshape(_NCHUNK, 128))
    wstack = W_c.T.reshape(H, DT, DM)
    out2d = _out_matmul(g3.reshape(L * NUM, H, 128), wstack,
                        b_c.reshape(1, DM))
    return out2d.reshape(1, L, NUM, DM)
